# Initial kernel scaffold; baseline (speedup 1.0000x reference)
import functools

import jax
import jax.numpy as jnp
from jax import lax
from jax.experimental import pallas as pl
from jax.experimental.pallas import tpu as pltpu
from jax.experimental.pallas import tpu_sc as plsc

_D = 60
_DP = 64
_W = 20
_EPS = 1e-12
_NC = 2
_NS = 16
_NW = _NC * _NS
_C = 128


def _prescale_body(x_ref, o_ref, *, scale):
    x = x_ref[...]
    n = jnp.sqrt(jnp.sum(x * x, axis=1, keepdims=True))
    y = x * (scale / jnp.maximum(n, _EPS))
    o_ref[...] = jnp.concatenate(
        [y, jnp.zeros((y.shape[0], _DP - _D), y.dtype)], axis=1)


def _prescale(table, scale, blk):
    n = table.shape[0]
    return pl.pallas_call(
        functools.partial(_prescale_body, scale=scale),
        grid=(n // blk,),
        in_specs=[pl.BlockSpec((blk, _D), lambda i: (i, 0))],
        out_specs=pl.BlockSpec((blk, _DP), lambda i: (i, 0)),
        out_shape=jax.ShapeDtypeStruct((n, _DP), jnp.float32),
    )(table)


def _sc_gather(B, head, relation, tail, words_t, zeros_c,
               entity_embedding, rel_s, proj_s, word_s, e_bias, r_bias):
    nb = B // _NW
    nchunks = nb // _C
    f32 = jnp.float32
    mesh = plsc.VectorSubcoreMesh(core_axis_name="c", subcore_axis_name="s")
    out_type = (
        jax.ShapeDtypeStruct((B, _D), f32),
        jax.ShapeDtypeStruct((B, _D), f32),
        jax.ShapeDtypeStruct((B, _DP), f32),
        jax.ShapeDtypeStruct((B, _DP), f32),
        jax.ShapeDtypeStruct((B, _DP), f32),
        jax.ShapeDtypeStruct((B, _DP), f32),
        jax.ShapeDtypeStruct((B, _DP), f32),
        jax.ShapeDtypeStruct((B, 1), f32),
        jax.ShapeDtypeStruct((B, 1), f32),
        jax.ShapeDtypeStruct((B, 1), f32),
    )
    scratch = [
        pltpu.VMEM((_C,), jnp.int32),
        pltpu.VMEM((_C,), jnp.int32),
        pltpu.VMEM((_C,), jnp.int32),
        pltpu.VMEM((3 * _W, _C), jnp.int32),
        pltpu.VMEM((_C, _D), f32),
        pltpu.VMEM((_C, _D), f32),
        pltpu.VMEM((_C, _DP), f32),
        pltpu.VMEM((_C, _DP), f32),
        pltpu.VMEM((_C, _DP), f32),
        pltpu.VMEM((_C, _DP), f32),
        pltpu.VMEM((_C, _DP), f32),
        pltpu.VMEM((_C, 1), f32),
        pltpu.VMEM((_C, 1), f32),
        pltpu.VMEM((_C, 1), f32),
        pltpu.SemaphoreType.DMA,
    ]

    @functools.partial(pl.kernel, out_type=out_type, mesh=mesh,
                       scratch_types=scratch)
    def k(head_r, rel_r, tail_r, wt_r, z_r, ent_r, rls_r, pjs_r, wds_r,
          eb_r, rb_r,
          o_eh, o_et, o_rl, o_pj, o_ah, o_ar, o_at, o_hb, o_tb, o_rb,
          hidx, tidx, ridx, widx, beh, bet, brl, bpj, bah, bar, bat,
          bhb, btb, brb, sem):
        wid = lax.axis_index("s") * _NC + lax.axis_index("c")
        base0 = wid * nb

        def chunk(ci, carry):
            base = base0 + ci * _C
            pltpu.sync_copy(head_r.at[pl.ds(base, _C)], hidx)
            pltpu.sync_copy(tail_r.at[pl.ds(base, _C)], tidx)
            pltpu.sync_copy(rel_r.at[pl.ds(base, _C)], ridx)
            pltpu.sync_copy(wt_r.at[:, pl.ds(base, _C)], widx)
            pltpu.sync_copy(z_r, bah)
            pltpu.sync_copy(z_r, bar)
            pltpu.sync_copy(z_r, bat)
            cps = []
            for s, accbuf in enumerate((bah, bar, bat)):
                for w in range(_W):
                    cps.append(pltpu.async_copy(
                        wds_r.at[widx.at[s * _W + w]], accbuf, sem,
                        add=True))
            cps.append(pltpu.async_copy(ent_r.at[hidx], beh, sem))
            cps.append(pltpu.async_copy(ent_r.at[tidx], bet, sem))
            cps.append(pltpu.async_copy(rls_r.at[ridx], brl, sem))
            cps.append(pltpu.async_copy(pjs_r.at[ridx], bpj, sem))
            cps.append(pltpu.async_copy(eb_r.at[hidx], bhb, sem))
            cps.append(pltpu.async_copy(eb_r.at[tidx], btb, sem))
            cps.append(pltpu.async_copy(rb_r.at[ridx], brb, sem))
            for cp in cps:
                cp.wait()
            pltpu.sync_copy(beh, o_eh.at[pl.ds(base, _C)])
            pltpu.sync_copy(bet, o_et.at[pl.ds(base, _C)])
            pltpu.sync_copy(brl, o_rl.at[pl.ds(base, _C)])
            pltpu.sync_copy(bpj, o_pj.at[pl.ds(base, _C)])
            pltpu.sync_copy(bah, o_ah.at[pl.ds(base, _C)])
            pltpu.sync_copy(bar, o_ar.at[pl.ds(base, _C)])
            pltpu.sync_copy(bat, o_at.at[pl.ds(base, _C)])
            pltpu.sync_copy(bhb, o_hb.at[pl.ds(base, _C)])
            pltpu.sync_copy(btb, o_tb.at[pl.ds(base, _C)])
            pltpu.sync_copy(brb, o_rb.at[pl.ds(base, _C)])
            return carry

        lax.fori_loop(0, nchunks, chunk, 0)

    return k(head, relation, tail, words_t, zeros_c, entity_embedding,
             rel_s, proj_s, word_s, e_bias, r_bias)


def _score_body(eh, et, rl, pj, ah, ar, at_, hb, tb, rb, o):
    def nrm(x):
        n = jnp.sqrt(jnp.sum(x * x, axis=1, keepdims=True))
        return x / jnp.maximum(n, _EPS)

    head_e = nrm(eh[...]) + ah[...][:, :_D]
    tail_e = nrm(et[...]) + at_[...][:, :_D]
    rel_e = rl[...][:, :_D] + ar[...][:, :_D]
    p = pj[...][:, :_D]
    hp = head_e - jnp.sum(p * head_e, axis=1, keepdims=True) * p
    tp = tail_e - jnp.sum(p * tail_e, axis=1, keepdims=True) * p
    diff = hp + rel_e - tp
    sc = -jnp.sqrt(jnp.sum(diff * diff, axis=1))
    o[...] = sc + hb[...][:, 0] + tb[...][:, 0] + rb[...][:, 0]


def _score(B, eh, et, rl, pj, ah, ar, at_, hb, tb, rb):
    blk = 2048
    specD = pl.BlockSpec((blk, _D), lambda i: (i, 0))
    specP = pl.BlockSpec((blk, _DP), lambda i: (i, 0))
    spec1 = pl.BlockSpec((blk, 1), lambda i: (i, 0))
    return pl.pallas_call(
        _score_body,
        grid=(B // blk,),
        in_specs=[specD, specD, specP, specP, specP, specP, specP,
                  spec1, spec1, spec1],
        out_specs=pl.BlockSpec((blk,), lambda i: (i,)),
        out_shape=jax.ShapeDtypeStruct((B,), jnp.float32),
    )(eh, et, rl, pj, ah, ar, at_, hb, tb, rb)


def kernel(head, relation, tail, head_w, rel_w, tail_w, entity_embedding,
           relation_embedding, word_embedding, e_bias, r_bias,
           relation_projection):
    B = head.shape[0]
    i32 = jnp.int32
    words_t = jnp.concatenate(
        [head_w.T.astype(i32), rel_w.T.astype(i32), tail_w.T.astype(i32)],
        axis=0)
    word_s = _prescale(word_embedding, 1.0 / _W, 2000)
    rel_s = _prescale(relation_embedding, 1.0, 1000)
    proj_s = _prescale(relation_projection, 1.0, 1000)
    zeros_c = jnp.zeros((_C, _DP), jnp.float32)
    outs = _sc_gather(B, head.astype(i32), relation.astype(i32),
                      tail.astype(i32), words_t, zeros_c, entity_embedding,
                      rel_s, proj_s, word_s, e_bias, r_bias)
    return _score(B, *outs)


# speedup vs baseline: 4.1698x; 4.1698x over previous
import functools

import jax
import jax.numpy as jnp
from jax import lax
from jax.experimental import pallas as pl
from jax.experimental.pallas import tpu as pltpu
from jax.experimental.pallas import tpu_sc as plsc

_D = 60
_DP = 64
_DE = 80
_W = 20
_EPS = 1e-12
_NC = 2
_NS = 16
_NW = _NC * _NS
_C = 128
_EV = 16


def _prescale_body(x_ref, o_ref, *, scale):
    x = x_ref[...]
    n = jnp.sqrt(jnp.sum(x * x, axis=1, keepdims=True))
    y = x * (scale / jnp.maximum(n, _EPS))
    o_ref[...] = jnp.concatenate(
        [y, jnp.zeros((y.shape[0], _DP - _D), y.dtype)], axis=1)


def _prescale(table, scale, blk):
    n = table.shape[0]
    return pl.pallas_call(
        functools.partial(_prescale_body, scale=scale),
        grid=(n // blk,),
        in_specs=[pl.BlockSpec((blk, _D), lambda i: (i, 0))],
        out_specs=pl.BlockSpec((blk, _DP), lambda i: (i, 0)),
        out_shape=jax.ShapeDtypeStruct((n, _DP), jnp.float32),
    )(table)


def _sc_gather(B, relation, idxs, words_t, zeros_c,
               ent16, rel_s, proj_s, word_s, ebias16, rbias16):
    nb = B // _NW
    nchunks = nb // _C
    f32 = jnp.float32
    mesh = plsc.VectorSubcoreMesh(core_axis_name="c", subcore_axis_name="s")
    out_type = (
        jax.ShapeDtypeStruct((B, _DE), f32),
        jax.ShapeDtypeStruct((B, _DE), f32),
        jax.ShapeDtypeStruct((B, _DP), f32),
        jax.ShapeDtypeStruct((B, _DP), f32),
        jax.ShapeDtypeStruct((B, _DP), f32),
        jax.ShapeDtypeStruct((B, _DP), f32),
        jax.ShapeDtypeStruct((B, _DP), f32),
        jax.ShapeDtypeStruct((B, _EV), f32),
        jax.ShapeDtypeStruct((B, _EV), f32),
        jax.ShapeDtypeStruct((B, _EV), f32),
    )
    scratch = [
        pltpu.VMEM((_C,), jnp.int32),
        pltpu.VMEM((13, _C), jnp.int32),
        pltpu.VMEM((3 * _W, _C), jnp.int32),
        pltpu.VMEM((5, _C, _EV), f32),
        pltpu.VMEM((5, _C, _EV), f32),
        pltpu.VMEM((_C, _DP), f32),
        pltpu.VMEM((_C, _DP), f32),
        pltpu.VMEM((_C, _DP), f32),
        pltpu.VMEM((_C, _DP), f32),
        pltpu.VMEM((_C, _DP), f32),
        pltpu.VMEM((_C, _EV), f32),
        pltpu.VMEM((_C, _EV), f32),
        pltpu.VMEM((_C, _EV), f32),
        pltpu.SemaphoreType.DMA,
    ]

    @functools.partial(
        pl.kernel, out_type=out_type, mesh=mesh, scratch_types=scratch,
        compiler_params=pltpu.CompilerParams(use_tc_tiling_on_sc=False))
    def k(rel_r, idxs_r, wt_r, z_r, ent_r, rls_r, pjs_r, wds_r,
          eb_r, rb_r,
          o_eh, o_et, o_rl, o_pj, o_ah, o_ar, o_at, o_hb, o_tb, o_rb,
          ridx, sidx, widx, beh, bet, brl, bpj, bah, bar, bat,
          bhb, btb, brb, sem):
        wid = lax.axis_index("s") * _NC + lax.axis_index("c")
        base0 = wid * nb

        def chunk(ci, carry):
            base = base0 + ci * _C
            pltpu.sync_copy(rel_r.at[pl.ds(base, _C)], ridx)
            pltpu.sync_copy(idxs_r.at[:, pl.ds(base, _C)], sidx)
            pltpu.sync_copy(wt_r.at[:, pl.ds(base, _C)], widx)
            pltpu.sync_copy(z_r, bah)
            pltpu.sync_copy(z_r, bar)
            pltpu.sync_copy(z_r, bat)
            cps = []
            for s, accbuf in enumerate((bah, bar, bat)):
                for w in range(_W):
                    cps.append(pltpu.async_copy(
                        wds_r.at[widx.at[s * _W + w]], accbuf, sem,
                        add=True))
            for j in range(5):
                cps.append(pltpu.async_copy(
                    ent_r.at[sidx.at[j]], beh.at[j], sem))
                cps.append(pltpu.async_copy(
                    ent_r.at[sidx.at[5 + j]], bet.at[j], sem))
            cps.append(pltpu.async_copy(rls_r.at[ridx], brl, sem))
            cps.append(pltpu.async_copy(pjs_r.at[ridx], bpj, sem))
            cps.append(pltpu.async_copy(eb_r.at[sidx.at[10]], bhb, sem))
            cps.append(pltpu.async_copy(eb_r.at[sidx.at[11]], btb, sem))
            cps.append(pltpu.async_copy(rb_r.at[sidx.at[12]], brb, sem))
            for cp in cps:
                cp.wait()
            for j in range(5):
                pltpu.sync_copy(
                    beh.at[j], o_eh.at[pl.ds(base, _C), pl.ds(_EV * j, _EV)])
                pltpu.sync_copy(
                    bet.at[j], o_et.at[pl.ds(base, _C), pl.ds(_EV * j, _EV)])
            pltpu.sync_copy(brl, o_rl.at[pl.ds(base, _C)])
            pltpu.sync_copy(bpj, o_pj.at[pl.ds(base, _C)])
            pltpu.sync_copy(bah, o_ah.at[pl.ds(base, _C)])
            pltpu.sync_copy(bar, o_ar.at[pl.ds(base, _C)])
            pltpu.sync_copy(bat, o_at.at[pl.ds(base, _C)])
            pltpu.sync_copy(bhb, o_hb.at[pl.ds(base, _C)])
            pltpu.sync_copy(btb, o_tb.at[pl.ds(base, _C)])
            pltpu.sync_copy(brb, o_rb.at[pl.ds(base, _C)])
            return carry

        lax.fori_loop(0, nchunks, chunk, 0)

    return k(relation, idxs, words_t, zeros_c, ent16,
             rel_s, proj_s, word_s, ebias16, rbias16)


def _score_body(eh, et, rl, pj, ah, ar, at_, hb, tb, rb, aux, o):
    a = aux[...]
    blk = a.shape[0]
    iota16 = lax.broadcasted_iota(jnp.int32, (blk, _EV), 1)

    def window(x, off):
        return jnp.where(
            off == 0, x[:, 0:_D],
            jnp.where(off == 4, x[:, 4:4 + _D],
                      jnp.where(off == 8, x[:, 8:8 + _D], x[:, 12:12 + _D])))

    def lane_pick(x, lane):
        return jnp.sum(jnp.where(iota16 == lane, x, 0.0), axis=1)

    def nrm(x):
        n = jnp.sqrt(jnp.sum(x * x, axis=1, keepdims=True))
        return x / jnp.maximum(n, _EPS)

    eh60 = window(eh[...], a[:, 0:1])
    et60 = window(et[...], a[:, 1:2])
    head_e = nrm(eh60) + ah[...][:, :_D]
    tail_e = nrm(et60) + at_[...][:, :_D]
    rel_e = rl[...][:, :_D] + ar[...][:, :_D]
    p = pj[...][:, :_D]
    hp = head_e - jnp.sum(p * head_e, axis=1, keepdims=True) * p
    tp = tail_e - jnp.sum(p * tail_e, axis=1, keepdims=True) * p
    diff = hp + rel_e - tp
    sc = -jnp.sqrt(jnp.sum(diff * diff, axis=1))
    o[...] = (sc + lane_pick(hb[...], a[:, 2:3])
              + lane_pick(tb[...], a[:, 3:4])
              + lane_pick(rb[...], a[:, 4:5]))


def _score(B, eh, et, rl, pj, ah, ar, at_, hb, tb, rb, aux):
    blk = 2048
    specE = pl.BlockSpec((blk, _DE), lambda i: (i, 0))
    specP = pl.BlockSpec((blk, _DP), lambda i: (i, 0))
    specV = pl.BlockSpec((blk, _EV), lambda i: (i, 0))
    specA = pl.BlockSpec((blk, 8), lambda i: (i, 0))
    return pl.pallas_call(
        _score_body,
        grid=(B // blk,),
        in_specs=[specE, specE, specP, specP, specP, specP, specP,
                  specV, specV, specV, specA],
        out_specs=pl.BlockSpec((blk,), lambda i: (i,)),
        out_shape=jax.ShapeDtypeStruct((B,), jnp.float32),
    )(eh, et, rl, pj, ah, ar, at_, hb, tb, rb, aux)


def kernel(head, relation, tail, head_w, rel_w, tail_w, entity_embedding,
           relation_embedding, word_embedding, e_bias, r_bias,
           relation_projection):
    B = head.shape[0]
    i32 = jnp.int32
    head = head.astype(i32)
    tail = tail.astype(i32)
    relation = relation.astype(i32)
    nent, _ = entity_embedding.shape
    nview = nent * _D // _EV

    words_t = jnp.concatenate(
        [head_w.T.astype(i32), rel_w.T.astype(i32), tail_w.T.astype(i32)],
        axis=0)
    j5 = jnp.arange(5, dtype=i32)[:, None]
    hvb = (head * (_D // 4)) // 4
    tvb = (tail * (_D // 4)) // 4
    idxs = jnp.concatenate([
        jnp.minimum(hvb[None, :] + j5, nview - 1),
        jnp.minimum(tvb[None, :] + j5, nview - 1),
        (head >> 4)[None, :],
        (tail >> 4)[None, :],
        (relation >> 4)[None, :],
    ], axis=0)
    aux = jnp.stack([
        head * _D - hvb * _EV, tail * _D - tvb * _EV,
        head & (_EV - 1), tail & (_EV - 1), relation & (_EV - 1),
        jnp.zeros_like(head), jnp.zeros_like(head), jnp.zeros_like(head),
    ], axis=1)

    word_s = _prescale(word_embedding, 1.0 / _W, 2000)
    rel_s = _prescale(relation_embedding, 1.0, 1000)
    proj_s = _prescale(relation_projection, 1.0, 1000)
    zeros_c = jnp.zeros((_C, _DP), jnp.float32)

    ent16 = entity_embedding.reshape(nview, _EV)
    ebias16 = e_bias.reshape(-1, _EV)
    nr = r_bias.shape[0]
    rpad = (-nr) % _EV
    rbias16 = jnp.concatenate(
        [r_bias, jnp.zeros((rpad, 1), r_bias.dtype)], axis=0).reshape(-1, _EV)

    outs = _sc_gather(B, relation, idxs, words_t, zeros_c, ent16,
                      rel_s, proj_s, word_s, ebias16, rbias16)
    return _score(B, *outs, aux)


# speedup vs baseline: 4.2783x; 1.0260x over previous
import functools

import jax
import jax.numpy as jnp
from jax import lax
from jax.experimental import pallas as pl
from jax.experimental.pallas import tpu as pltpu
from jax.experimental.pallas import tpu_sc as plsc

_D = 60
_DP = 64
_W = 20
_EPS = 1e-12
_NC = 2
_NS = 16
_NW = _NC * _NS
_C = 128
_EV = 16
_LW = 128


def _prescale_body(x_ref, o_ref, *, scale):
    x = x_ref[...]
    n = jnp.sqrt(jnp.sum(x * x, axis=1, keepdims=True))
    y = x * (scale / jnp.maximum(n, _EPS))
    o_ref[...] = jnp.concatenate(
        [y, jnp.zeros((y.shape[0], _DP - _D), y.dtype)], axis=1)


def _prescale(table, scale, blk):
    n = table.shape[0]
    return pl.pallas_call(
        functools.partial(_prescale_body, scale=scale),
        grid=(n // blk,),
        in_specs=[pl.BlockSpec((blk, _D), lambda i: (i, 0))],
        out_specs=pl.BlockSpec((blk, _DP), lambda i: (i, 0)),
        out_shape=jax.ShapeDtypeStruct((n, _DP), jnp.float32),
    )(table)


def _sc_words(B, relation, idxs, words_t, zeros_c,
              rel_s, proj_s, word_s, ebias16, rbias16):
    nb = B // _NW
    nchunks = nb // _C
    f32 = jnp.float32
    mesh = plsc.VectorSubcoreMesh(core_axis_name="c", subcore_axis_name="s")
    out_type = (
        jax.ShapeDtypeStruct((B, _DP), f32),
        jax.ShapeDtypeStruct((B, _DP), f32),
        jax.ShapeDtypeStruct((B, _DP), f32),
        jax.ShapeDtypeStruct((B, _DP), f32),
        jax.ShapeDtypeStruct((B, _DP), f32),
        jax.ShapeDtypeStruct((B, _EV), f32),
        jax.ShapeDtypeStruct((B, _EV), f32),
        jax.ShapeDtypeStruct((B, _EV), f32),
    )
    scratch = [
        pltpu.VMEM((_C,), jnp.int32),
        pltpu.VMEM((3, _C), jnp.int32),
        pltpu.VMEM((3 * _W, _C), jnp.int32),
        pltpu.VMEM((_C, _DP), f32),
        pltpu.VMEM((_C, _DP), f32),
        pltpu.VMEM((_C, _DP), f32),
        pltpu.VMEM((_C, _DP), f32),
        pltpu.VMEM((_C, _DP), f32),
        pltpu.VMEM((_C, _EV), f32),
        pltpu.VMEM((_C, _EV), f32),
        pltpu.VMEM((_C, _EV), f32),
        pltpu.SemaphoreType.DMA,
    ]

    @functools.partial(
        pl.kernel, out_type=out_type, mesh=mesh, scratch_types=scratch,
        compiler_params=pltpu.CompilerParams(use_tc_tiling_on_sc=False))
    def k(rel_r, idxs_r, wt_r, z_r, rls_r, pjs_r, wds_r, eb_r, rb_r,
          o_rl, o_pj, o_ah, o_ar, o_at, o_hb, o_tb, o_rb,
          ridx, sidx, widx, brl, bpj, bah, bar, bat, bhb, btb, brb, sem):
        wid = lax.axis_index("s") * _NC + lax.axis_index("c")
        base0 = wid * nb

        def chunk(ci, carry):
            base = base0 + ci * _C
            pltpu.sync_copy(rel_r.at[pl.ds(base, _C)], ridx)
            pltpu.sync_copy(idxs_r.at[:, pl.ds(base, _C)], sidx)
            pltpu.sync_copy(wt_r.at[:, pl.ds(base, _C)], widx)
            pltpu.sync_copy(z_r, bah)
            pltpu.sync_copy(z_r, bar)
            pltpu.sync_copy(z_r, bat)
            cps = []
            for s, accbuf in enumerate((bah, bar, bat)):
                for w in range(_W):
                    cps.append(pltpu.async_copy(
                        wds_r.at[widx.at[s * _W + w]], accbuf, sem,
                        add=True))
            cps.append(pltpu.async_copy(rls_r.at[ridx], brl, sem))
            cps.append(pltpu.async_copy(pjs_r.at[ridx], bpj, sem))
            cps.append(pltpu.async_copy(eb_r.at[sidx.at[0]], bhb, sem))
            cps.append(pltpu.async_copy(eb_r.at[sidx.at[1]], btb, sem))
            cps.append(pltpu.async_copy(rb_r.at[sidx.at[2]], brb, sem))
            for cp in cps:
                cp.wait()
            pltpu.sync_copy(brl, o_rl.at[pl.ds(base, _C)])
            pltpu.sync_copy(bpj, o_pj.at[pl.ds(base, _C)])
            pltpu.sync_copy(bah, o_ah.at[pl.ds(base, _C)])
            pltpu.sync_copy(bar, o_ar.at[pl.ds(base, _C)])
            pltpu.sync_copy(bat, o_at.at[pl.ds(base, _C)])
            pltpu.sync_copy(bhb, o_hb.at[pl.ds(base, _C)])
            pltpu.sync_copy(btb, o_tb.at[pl.ds(base, _C)])
            pltpu.sync_copy(brb, o_rb.at[pl.ds(base, _C)])
            return carry

        lax.fori_loop(0, nchunks, chunk, 0)

    return k(relation, idxs, words_t, zeros_c,
             rel_s, proj_s, word_s, ebias16, rbias16)


def _sc_entity(B, idxs, ent128):
    nb = B // _NW
    nchunks = nb // _C
    f32 = jnp.float32
    mesh = plsc.VectorSubcoreMesh(core_axis_name="c", subcore_axis_name="s")
    out_type = (
        jax.ShapeDtypeStruct((B, 2 * _LW), f32),
        jax.ShapeDtypeStruct((B, 2 * _LW), f32),
    )
    scratch = [
        pltpu.VMEM((4, _C), jnp.int32),
        pltpu.VMEM((2, _C, _LW), f32),
        pltpu.VMEM((2, _C, _LW), f32),
        pltpu.SemaphoreType.DMA,
    ]

    @functools.partial(pl.kernel, out_type=out_type, mesh=mesh,
                       scratch_types=scratch)
    def k(idxs_r, ent_r, o_eh, o_et, sidx, beh, bet, sem):
        wid = lax.axis_index("s") * _NC + lax.axis_index("c")
        base0 = wid * nb

        def chunk(ci, carry):
            base = base0 + ci * _C
            pltpu.sync_copy(idxs_r.at[:, pl.ds(base, _C)], sidx)
            cps = []
            for j in range(2):
                cps.append(pltpu.async_copy(
                    ent_r.at[sidx.at[j]], beh.at[j], sem))
                cps.append(pltpu.async_copy(
                    ent_r.at[sidx.at[2 + j]], bet.at[j], sem))
            for cp in cps:
                cp.wait()
            for j in range(2):
                pltpu.sync_copy(
                    beh.at[j],
                    o_eh.at[pl.ds(base, _C), pl.ds(_LW * j, _LW)])
                pltpu.sync_copy(
                    bet.at[j],
                    o_et.at[pl.ds(base, _C), pl.ds(_LW * j, _LW)])
            return carry

        lax.fori_loop(0, nchunks, chunk, 0)

    return k(idxs, ent128)


def _score_body(eh, et, rl, pj, ah, ar, at_, hb, tb, rb, aux, o):
    a = aux[...]
    blk = a.shape[0]
    iota16 = lax.broadcasted_iota(jnp.int32, (blk, _EV), 1)

    def window(x, off):
        c = off >> 4
        f = off & 15
        y = x[:, 112:112 + 76]
        for ci in range(6, -1, -1):
            y = jnp.where(c == ci, x[:, 16 * ci:16 * ci + 76], y)
        return jnp.where(
            f == 0, y[:, 0:_D],
            jnp.where(f == 4, y[:, 4:4 + _D],
                      jnp.where(f == 8, y[:, 8:8 + _D], y[:, 12:12 + _D])))

    def lane_pick(x, lane):
        return jnp.sum(jnp.where(iota16 == lane, x, 0.0), axis=1)

    def nrm(x):
        n = jnp.sqrt(jnp.sum(x * x, axis=1, keepdims=True))
        return x / jnp.maximum(n, _EPS)

    eh60 = window(eh[...], a[:, 0:1])
    et60 = window(et[...], a[:, 1:2])
    head_e = nrm(eh60) + ah[...][:, :_D]
    tail_e = nrm(et60) + at_[...][:, :_D]
    rel_e = rl[...][:, :_D] + ar[...][:, :_D]
    p = pj[...][:, :_D]
    hp = head_e - jnp.sum(p * head_e, axis=1, keepdims=True) * p
    tp = tail_e - jnp.sum(p * tail_e, axis=1, keepdims=True) * p
    diff = hp + rel_e - tp
    sc = -jnp.sqrt(jnp.sum(diff * diff, axis=1))
    o[...] = (sc + lane_pick(hb[...], a[:, 2:3])
              + lane_pick(tb[...], a[:, 3:4])
              + lane_pick(rb[...], a[:, 4:5]))


def _score(B, eh, et, rl, pj, ah, ar, at_, hb, tb, rb, aux):
    blk = 2048
    specE = pl.BlockSpec((blk, 2 * _LW), lambda i: (i, 0))
    specP = pl.BlockSpec((blk, _DP), lambda i: (i, 0))
    specV = pl.BlockSpec((blk, _EV), lambda i: (i, 0))
    specA = pl.BlockSpec((blk, 8), lambda i: (i, 0))
    return pl.pallas_call(
        _score_body,
        grid=(B // blk,),
        in_specs=[specE, specE, specP, specP, specP, specP, specP,
                  specV, specV, specV, specA],
        out_specs=pl.BlockSpec((blk,), lambda i: (i,)),
        out_shape=jax.ShapeDtypeStruct((B,), jnp.float32),
    )(eh, et, rl, pj, ah, ar, at_, hb, tb, rb, aux)


def kernel(head, relation, tail, head_w, rel_w, tail_w, entity_embedding,
           relation_embedding, word_embedding, e_bias, r_bias,
           relation_projection):
    B = head.shape[0]
    i32 = jnp.int32
    head = head.astype(i32)
    tail = tail.astype(i32)
    relation = relation.astype(i32)
    nent, _ = entity_embedding.shape
    nview = nent * _D // _LW

    words_t = jnp.concatenate(
        [head_w.T.astype(i32), rel_w.T.astype(i32), tail_w.T.astype(i32)],
        axis=0)
    hvb = (head * (_D // 4)) // (_LW // 4)
    tvb = (tail * (_D // 4)) // (_LW // 4)
    idxs_e = jnp.stack([
        hvb, jnp.minimum(hvb + 1, nview - 1),
        tvb, jnp.minimum(tvb + 1, nview - 1),
    ], axis=0)
    idxs_w = jnp.stack([head >> 4, tail >> 4, relation >> 4], axis=0)
    aux = jnp.stack([
        head * _D - hvb * _LW, tail * _D - tvb * _LW,
        head & (_EV - 1), tail & (_EV - 1), relation & (_EV - 1),
        jnp.zeros_like(head), jnp.zeros_like(head), jnp.zeros_like(head),
    ], axis=1)

    word_s = _prescale(word_embedding, 1.0 / _W, 2000)
    rel_s = _prescale(relation_embedding, 1.0, 1000)
    proj_s = _prescale(relation_projection, 1.0, 1000)
    zeros_c = jnp.zeros((_C, _DP), jnp.float32)

    ebias16 = e_bias.reshape(-1, _EV)
    nr = r_bias.shape[0]
    rpad = (-nr) % _EV
    rbias16 = jnp.concatenate(
        [r_bias, jnp.zeros((rpad, 1), r_bias.dtype)], axis=0).reshape(-1, _EV)
    ent128 = entity_embedding.reshape(nview, _LW)

    rl, pj, ah, ar, at_, hb, tb, rb = _sc_words(
        B, relation, idxs_w, words_t, zeros_c,
        rel_s, proj_s, word_s, ebias16, rbias16)
    eh, et = _sc_entity(B, idxs_e, ent128)
    return _score(B, eh, et, rl, pj, ah, ar, at_, hb, tb, rb, aux)


# speedup vs baseline: 4.7591x; 1.1124x over previous
import functools

import jax
import jax.numpy as jnp
from jax import lax
from jax.experimental import pallas as pl
from jax.experimental.pallas import tpu as pltpu
from jax.experimental.pallas import tpu_sc as plsc

_D = 60
_DP = 64
_W = 20
_EPS = 1e-12
_NC = 2
_NS = 16
_NW = _NC * _NS
_C = 128
_EV = 16
_LW = 128


def _prescale_body(x_ref, o_ref, *, scale):
    x = x_ref[...]
    n = jnp.sqrt(jnp.sum(x * x, axis=1, keepdims=True))
    y = x * (scale / jnp.maximum(n, _EPS))
    o_ref[...] = jnp.concatenate(
        [y, jnp.zeros((y.shape[0], _DP - _D), y.dtype)], axis=1)


def _prescale(table, scale, blk):
    n = table.shape[0]
    return pl.pallas_call(
        functools.partial(_prescale_body, scale=scale),
        grid=(n // blk,),
        in_specs=[pl.BlockSpec((blk, _D), lambda i: (i, 0))],
        out_specs=pl.BlockSpec((blk, _DP), lambda i: (i, 0)),
        out_shape=jax.ShapeDtypeStruct((n, _DP), jnp.float32),
    )(table)


def _sc_words(B, relation, idxs, words_t, zeros_c,
              rel_s, proj_s, word_s, ebias16, rbias16):
    nb = B // _NW
    nchunks = nb // _C
    f32 = jnp.float32
    mesh = plsc.VectorSubcoreMesh(core_axis_name="c", subcore_axis_name="s")
    out_type = (
        jax.ShapeDtypeStruct((B, _DP), f32),
        jax.ShapeDtypeStruct((B, _DP), f32),
        jax.ShapeDtypeStruct((B, _DP), f32),
        jax.ShapeDtypeStruct((B, _DP), f32),
        jax.ShapeDtypeStruct((B, _DP), f32),
        jax.ShapeDtypeStruct((B, _EV), f32),
        jax.ShapeDtypeStruct((B, _EV), f32),
        jax.ShapeDtypeStruct((B, _EV), f32),
    )
    scratch = [
        pltpu.VMEM((_C,), jnp.int32),
        pltpu.VMEM((3, _C), jnp.int32),
        pltpu.VMEM((3 * _W, _C), jnp.int32),
        pltpu.VMEM((_C, _DP), f32),
        pltpu.VMEM((_C, _DP), f32),
        pltpu.VMEM((_C, _DP), f32),
        pltpu.VMEM((_C, _DP), f32),
        pltpu.VMEM((_C, _DP), f32),
        pltpu.VMEM((_C, _EV), f32),
        pltpu.VMEM((_C, _EV), f32),
        pltpu.VMEM((_C, _EV), f32),
        pltpu.SemaphoreType.DMA,
    ]

    @functools.partial(
        pl.kernel, out_type=out_type, mesh=mesh, scratch_types=scratch,
        compiler_params=pltpu.CompilerParams(use_tc_tiling_on_sc=False))
    def k(rel_r, idxs_r, wt_r, z_r, rls_r, pjs_r, wds_r, eb_r, rb_r,
          o_rl, o_pj, o_ah, o_ar, o_at, o_hb, o_tb, o_rb,
          ridx, sidx, widx, brl, bpj, bah, bar, bat, bhb, btb, brb, sem):
        wid = lax.axis_index("s") * _NC + lax.axis_index("c")
        base0 = wid * nb

        def chunk(ci, carry):
            base = base0 + ci * _C
            pltpu.sync_copy(rel_r.at[pl.ds(base, _C)], ridx)
            pltpu.sync_copy(idxs_r.at[:, pl.ds(base, _C)], sidx)
            pltpu.sync_copy(wt_r.at[:, pl.ds(base, _C)], widx)
            pltpu.sync_copy(z_r, bah)
            pltpu.sync_copy(z_r, bar)
            pltpu.sync_copy(z_r, bat)
            cps = []
            for s, accbuf in enumerate((bah, bar, bat)):
                for w in range(_W):
                    cps.append(pltpu.async_copy(
                        wds_r.at[widx.at[s * _W + w]], accbuf, sem,
                        add=True))
            cps.append(pltpu.async_copy(rls_r.at[ridx], brl, sem))
            cps.append(pltpu.async_copy(pjs_r.at[ridx], bpj, sem))
            cps.append(pltpu.async_copy(eb_r.at[sidx.at[0]], bhb, sem))
            cps.append(pltpu.async_copy(eb_r.at[sidx.at[1]], btb, sem))
            cps.append(pltpu.async_copy(rb_r.at[sidx.at[2]], brb, sem))
            for cp in cps:
                cp.wait()
            pltpu.sync_copy(brl, o_rl.at[pl.ds(base, _C)])
            pltpu.sync_copy(bpj, o_pj.at[pl.ds(base, _C)])
            pltpu.sync_copy(bah, o_ah.at[pl.ds(base, _C)])
            pltpu.sync_copy(bar, o_ar.at[pl.ds(base, _C)])
            pltpu.sync_copy(bat, o_at.at[pl.ds(base, _C)])
            pltpu.sync_copy(bhb, o_hb.at[pl.ds(base, _C)])
            pltpu.sync_copy(btb, o_tb.at[pl.ds(base, _C)])
            pltpu.sync_copy(brb, o_rb.at[pl.ds(base, _C)])
            return carry

        lax.fori_loop(0, nchunks, chunk, 0)

    return k(relation, idxs, words_t, zeros_c,
             rel_s, proj_s, word_s, ebias16, rbias16)


def _entpad_body(x_ref, o_ref):
    x = x_ref[...]
    n = jnp.sqrt(jnp.sum(x * x, axis=1, keepdims=True))
    y = x / jnp.maximum(n, _EPS)
    o_ref[...] = jnp.concatenate(
        [y, jnp.zeros((y.shape[0], _LW - _D), y.dtype)], axis=1)


def _entpad(table, blk):
    n = table.shape[0]
    return pl.pallas_call(
        _entpad_body,
        grid=(n // blk,),
        in_specs=[pl.BlockSpec((blk, _D), lambda i: (i, 0))],
        out_specs=pl.BlockSpec((blk, _LW), lambda i: (i, 0)),
        out_shape=jax.ShapeDtypeStruct((n, _LW), jnp.float32),
    )(table)


def _sc_entity(B, idxs, ent128):
    nb = B // _NW
    nchunks = nb // _C
    f32 = jnp.float32
    mesh = plsc.VectorSubcoreMesh(core_axis_name="c", subcore_axis_name="s")
    out_type = (
        jax.ShapeDtypeStruct((B, _LW), f32),
        jax.ShapeDtypeStruct((B, _LW), f32),
    )
    scratch = [
        pltpu.VMEM((2, _C), jnp.int32),
        pltpu.VMEM((_C, _LW), f32),
        pltpu.VMEM((_C, _LW), f32),
        pltpu.SemaphoreType.DMA,
    ]

    @functools.partial(pl.kernel, out_type=out_type, mesh=mesh,
                       scratch_types=scratch)
    def k(idxs_r, ent_r, o_eh, o_et, sidx, beh, bet, sem):
        wid = lax.axis_index("s") * _NC + lax.axis_index("c")
        base0 = wid * nb

        def chunk(ci, carry):
            base = base0 + ci * _C
            pltpu.sync_copy(idxs_r.at[:, pl.ds(base, _C)], sidx)
            cps = [
                pltpu.async_copy(ent_r.at[sidx.at[0]], beh, sem),
                pltpu.async_copy(ent_r.at[sidx.at[1]], bet, sem),
            ]
            for cp in cps:
                cp.wait()
            pltpu.sync_copy(beh, o_eh.at[pl.ds(base, _C)])
            pltpu.sync_copy(bet, o_et.at[pl.ds(base, _C)])
            return carry

        lax.fori_loop(0, nchunks, chunk, 0)

    return k(idxs, ent128)


def _score_body(eh, et, rl, pj, ah, ar, at_, hb, tb, rb, aux, o):
    a = aux[...]
    blk = a.shape[0]
    iota16 = lax.broadcasted_iota(jnp.int32, (blk, _EV), 1)

    def lane_pick(x, lane):
        return jnp.sum(jnp.where(iota16 == lane, x, 0.0), axis=1)

    head_e = eh[...][:, :_D] + ah[...][:, :_D]
    tail_e = et[...][:, :_D] + at_[...][:, :_D]
    rel_e = rl[...][:, :_D] + ar[...][:, :_D]
    p = pj[...][:, :_D]
    hp = head_e - jnp.sum(p * head_e, axis=1, keepdims=True) * p
    tp = tail_e - jnp.sum(p * tail_e, axis=1, keepdims=True) * p
    diff = hp + rel_e - tp
    sc = -jnp.sqrt(jnp.sum(diff * diff, axis=1))
    o[...] = (sc + lane_pick(hb[...], a[:, 0:1])
              + lane_pick(tb[...], a[:, 1:2])
              + lane_pick(rb[...], a[:, 2:3]))


def _score(B, eh, et, rl, pj, ah, ar, at_, hb, tb, rb, aux):
    blk = 2048
    specE = pl.BlockSpec((blk, _LW), lambda i: (i, 0))
    specP = pl.BlockSpec((blk, _DP), lambda i: (i, 0))
    specV = pl.BlockSpec((blk, _EV), lambda i: (i, 0))
    specA = pl.BlockSpec((blk, 8), lambda i: (i, 0))
    return pl.pallas_call(
        _score_body,
        grid=(B // blk,),
        in_specs=[specE, specE, specP, specP, specP, specP, specP,
                  specV, specV, specV, specA],
        out_specs=pl.BlockSpec((blk,), lambda i: (i,)),
        out_shape=jax.ShapeDtypeStruct((B,), jnp.float32),
    )(eh, et, rl, pj, ah, ar, at_, hb, tb, rb, aux)


def kernel(head, relation, tail, head_w, rel_w, tail_w, entity_embedding,
           relation_embedding, word_embedding, e_bias, r_bias,
           relation_projection):
    B = head.shape[0]
    i32 = jnp.int32
    head = head.astype(i32)
    tail = tail.astype(i32)
    relation = relation.astype(i32)
    words_t = jnp.concatenate(
        [head_w.T.astype(i32), rel_w.T.astype(i32), tail_w.T.astype(i32)],
        axis=0)
    idxs_e = jnp.stack([head, tail], axis=0)
    idxs_w = jnp.stack([head >> 4, tail >> 4, relation >> 4], axis=0)
    aux = jnp.stack([
        head & (_EV - 1), tail & (_EV - 1), relation & (_EV - 1),
        jnp.zeros_like(head), jnp.zeros_like(head), jnp.zeros_like(head),
        jnp.zeros_like(head), jnp.zeros_like(head),
    ], axis=1)

    word_s = _prescale(word_embedding, 1.0 / _W, 2000)
    rel_s = _prescale(relation_embedding, 1.0, 1000)
    proj_s = _prescale(relation_projection, 1.0, 1000)
    zeros_c = jnp.zeros((_C, _DP), jnp.float32)

    ebias16 = e_bias.reshape(-1, _EV)
    nr = r_bias.shape[0]
    rpad = (-nr) % _EV
    rbias16 = jnp.concatenate(
        [r_bias, jnp.zeros((rpad, 1), r_bias.dtype)], axis=0).reshape(-1, _EV)
    ent128 = _entpad(entity_embedding, 8000)

    rl, pj, ah, ar, at_, hb, tb, rb = _sc_words(
        B, relation, idxs_w, words_t, zeros_c,
        rel_s, proj_s, word_s, ebias16, rbias16)
    eh, et = _sc_entity(B, idxs_e, ent128)
    return _score(B, eh, et, rl, pj, ah, ar, at_, hb, tb, rb, aux)


# speedup vs baseline: 7.0322x; 1.4776x over previous
import functools

import jax
import jax.numpy as jnp
from jax import lax
from jax.experimental import pallas as pl
from jax.experimental.pallas import tpu as pltpu
from jax.experimental.pallas import tpu_sc as plsc

_D = 60
_DP = 64
_W = 20
_EPS = 1e-12
_NC = 2
_NS = 16
_NW = _NC * _NS
_C = 128
_EV = 16
_LW = 128


def _prescale_body(x_ref, o_ref, *, scale):
    x = x_ref[...]
    n = jnp.sqrt(jnp.sum(x * x, axis=1, keepdims=True))
    y = x * (scale / jnp.maximum(n, _EPS))
    o_ref[...] = jnp.concatenate(
        [y, jnp.zeros((y.shape[0], _DP - _D), y.dtype)], axis=1)


def _prescale(table, scale, blk):
    n = table.shape[0]
    return pl.pallas_call(
        functools.partial(_prescale_body, scale=scale),
        grid=(n // blk,),
        in_specs=[pl.BlockSpec((blk, _D), lambda i: (i, 0))],
        out_specs=pl.BlockSpec((blk, _DP), lambda i: (i, 0)),
        out_shape=jax.ShapeDtypeStruct((n, _DP), jnp.float32),
    )(table)


def _sc_words(B, relation, idxs, words_t, zeros_c,
              rel_s, proj_s, word_s, ebias16, rbias16):
    nb = B // _NW
    nchunks = nb // _C
    f32 = jnp.float32
    mesh = plsc.VectorSubcoreMesh(core_axis_name="c", subcore_axis_name="s")
    out_type = (
        jax.ShapeDtypeStruct((B, _DP), f32),
        jax.ShapeDtypeStruct((B, _DP), f32),
        jax.ShapeDtypeStruct((B, _DP), f32),
        jax.ShapeDtypeStruct((B, _DP), f32),
        jax.ShapeDtypeStruct((B, _DP), f32),
        jax.ShapeDtypeStruct((B, _EV), f32),
        jax.ShapeDtypeStruct((B, _EV), f32),
        jax.ShapeDtypeStruct((B, _EV), f32),
    )
    scratch = [
        pltpu.VMEM((_C,), jnp.int32),
        pltpu.VMEM((3, _C), jnp.int32),
        pltpu.VMEM((3 * _W, _C), jnp.int32),
        pltpu.VMEM((_C, _DP), f32),
        pltpu.VMEM((_C, _DP), f32),
        pltpu.VMEM((_C, _DP), f32),
        pltpu.VMEM((_C, _DP), f32),
        pltpu.VMEM((_C, _DP), f32),
        pltpu.VMEM((_C, _EV), f32),
        pltpu.VMEM((_C, _EV), f32),
        pltpu.VMEM((_C, _EV), f32),
        pltpu.SemaphoreType.DMA,
    ]

    @functools.partial(
        pl.kernel, out_type=out_type, mesh=mesh, scratch_types=scratch,
        compiler_params=pltpu.CompilerParams(use_tc_tiling_on_sc=False))
    def k(rel_r, idxs_r, wt_r, z_r, rls_r, pjs_r, wds_r, eb_r, rb_r,
          o_rl, o_pj, o_ah, o_ar, o_at, o_hb, o_tb, o_rb,
          ridx, sidx, widx, brl, bpj, bah, bar, bat, bhb, btb, brb, sem):
        wid = lax.axis_index("s") * _NC + lax.axis_index("c")
        base0 = wid * nb

        def chunk(ci, carry):
            base = base0 + ci * _C
            pltpu.sync_copy(rel_r.at[pl.ds(base, _C)], ridx)
            pltpu.sync_copy(idxs_r.at[:, pl.ds(base, _C)], sidx)
            pltpu.sync_copy(wt_r.at[:, pl.ds(base, _C)], widx)
            pltpu.sync_copy(z_r, bah)
            pltpu.sync_copy(z_r, bar)
            pltpu.sync_copy(z_r, bat)
            cps = []
            for s, accbuf in enumerate((bah, bar, bat)):
                for w in range(_W):
                    cps.append(pltpu.async_copy(
                        wds_r.at[widx.at[s * _W + w]], accbuf, sem,
                        add=True))
            cps.append(pltpu.async_copy(rls_r.at[ridx], brl, sem))
            cps.append(pltpu.async_copy(pjs_r.at[ridx], bpj, sem))
            cps.append(pltpu.async_copy(eb_r.at[sidx.at[0]], bhb, sem))
            cps.append(pltpu.async_copy(eb_r.at[sidx.at[1]], btb, sem))
            cps.append(pltpu.async_copy(rb_r.at[sidx.at[2]], brb, sem))
            for cp in cps:
                cp.wait()
            pltpu.sync_copy(brl, o_rl.at[pl.ds(base, _C)])
            pltpu.sync_copy(bpj, o_pj.at[pl.ds(base, _C)])
            pltpu.sync_copy(bah, o_ah.at[pl.ds(base, _C)])
            pltpu.sync_copy(bar, o_ar.at[pl.ds(base, _C)])
            pltpu.sync_copy(bat, o_at.at[pl.ds(base, _C)])
            pltpu.sync_copy(bhb, o_hb.at[pl.ds(base, _C)])
            pltpu.sync_copy(btb, o_tb.at[pl.ds(base, _C)])
            pltpu.sync_copy(brb, o_rb.at[pl.ds(base, _C)])
            return carry

        lax.fori_loop(0, nchunks, chunk, 0)

    return k(relation, idxs, words_t, zeros_c,
             rel_s, proj_s, word_s, ebias16, rbias16)


def _entpad_body(x_ref, o_ref):
    x = x_ref[...]
    n = jnp.sqrt(jnp.sum(x * x, axis=1, keepdims=True))
    y = x / jnp.maximum(n, _EPS)
    o_ref[...] = jnp.concatenate(
        [y, jnp.zeros((y.shape[0], _LW - _D), y.dtype)], axis=1)


def _entpad(table, blk):
    n = table.shape[0]
    return pl.pallas_call(
        _entpad_body,
        grid=(n // blk,),
        in_specs=[pl.BlockSpec((blk, _D), lambda i: (i, 0))],
        out_specs=pl.BlockSpec((blk, _LW), lambda i: (i, 0)),
        out_shape=jax.ShapeDtypeStruct((n, _LW), jnp.float32),
    )(table)


_GS = 16


def _sc_entity(B, idxs, ent):
    nb = B // _NW
    nchunks = nb // _C
    f32 = jnp.float32
    mesh = plsc.VectorSubcoreMesh(core_axis_name="c", subcore_axis_name="s")
    out_type = (
        jax.ShapeDtypeStruct((B, _D), f32),
        jax.ShapeDtypeStruct((B, _D), f32),
    )
    scratch = [
        pltpu.VMEM((2, _C), jnp.int32),
        pltpu.VMEM((_C, _D), f32),
        pltpu.VMEM((_C, _D), f32),
        pltpu.SemaphoreType.DMA,
    ]

    @functools.partial(pl.kernel, out_type=out_type, mesh=mesh,
                       scratch_types=scratch)
    def k(idxs_r, ent_r, o_eh, o_et, sidx, beh, bet, sem):
        wid = lax.axis_index("s") * _NC + lax.axis_index("c")
        base0 = wid * nb

        def chunk(ci, carry):
            base = base0 + ci * _C
            pltpu.sync_copy(idxs_r.at[:, pl.ds(base, _C)], sidx)

            def group(gi, carry2):
                g0 = gi * _GS
                hvec = sidx[0, pl.ds(g0, _GS)]
                tvec = sidx[1, pl.ds(g0, _GS)]
                for i in range(_GS):
                    pltpu.async_copy(
                        ent_r.at[pl.ds(hvec[i], 1)],
                        beh.at[pl.ds(g0 + i, 1)], sem)
                    pltpu.async_copy(
                        ent_r.at[pl.ds(tvec[i], 1)],
                        bet.at[pl.ds(g0 + i, 1)], sem)
                pltpu.make_async_copy(
                    ent_r.at[pl.ds(0, _GS)], beh.at[pl.ds(g0, _GS)],
                    sem).wait()
                pltpu.make_async_copy(
                    ent_r.at[pl.ds(0, _GS)], bet.at[pl.ds(g0, _GS)],
                    sem).wait()
                return carry2

            lax.fori_loop(0, _C // _GS, group, 0)
            pltpu.sync_copy(beh, o_eh.at[pl.ds(base, _C)])
            pltpu.sync_copy(bet, o_et.at[pl.ds(base, _C)])
            return carry

        lax.fori_loop(0, nchunks, chunk, 0)

    return k(idxs, ent)


def _score_body(eh, et, rl, pj, ah, ar, at_, hb, tb, rb, aux, o):
    a = aux[...]
    blk = a.shape[0]
    iota16 = lax.broadcasted_iota(jnp.int32, (blk, _EV), 1)

    def lane_pick(x, lane):
        return jnp.sum(jnp.where(iota16 == lane, x, 0.0), axis=1)

    def nrm(x):
        n = jnp.sqrt(jnp.sum(x * x, axis=1, keepdims=True))
        return x / jnp.maximum(n, _EPS)

    head_e = nrm(eh[...]) + ah[...][:, :_D]
    tail_e = nrm(et[...]) + at_[...][:, :_D]
    rel_e = rl[...][:, :_D] + ar[...][:, :_D]
    p = pj[...][:, :_D]
    hp = head_e - jnp.sum(p * head_e, axis=1, keepdims=True) * p
    tp = tail_e - jnp.sum(p * tail_e, axis=1, keepdims=True) * p
    diff = hp + rel_e - tp
    sc = -jnp.sqrt(jnp.sum(diff * diff, axis=1))
    o[...] = (sc + lane_pick(hb[...], a[:, 0:1])
              + lane_pick(tb[...], a[:, 1:2])
              + lane_pick(rb[...], a[:, 2:3]))


def _score(B, eh, et, rl, pj, ah, ar, at_, hb, tb, rb, aux):
    blk = 2048
    specE = pl.BlockSpec((blk, _D), lambda i: (i, 0))
    specP = pl.BlockSpec((blk, _DP), lambda i: (i, 0))
    specV = pl.BlockSpec((blk, _EV), lambda i: (i, 0))
    specA = pl.BlockSpec((blk, 8), lambda i: (i, 0))
    return pl.pallas_call(
        _score_body,
        grid=(B // blk,),
        in_specs=[specE, specE, specP, specP, specP, specP, specP,
                  specV, specV, specV, specA],
        out_specs=pl.BlockSpec((blk,), lambda i: (i,)),
        out_shape=jax.ShapeDtypeStruct((B,), jnp.float32),
    )(eh, et, rl, pj, ah, ar, at_, hb, tb, rb, aux)


def kernel(head, relation, tail, head_w, rel_w, tail_w, entity_embedding,
           relation_embedding, word_embedding, e_bias, r_bias,
           relation_projection):
    B = head.shape[0]
    i32 = jnp.int32
    head = head.astype(i32)
    tail = tail.astype(i32)
    relation = relation.astype(i32)
    words_t = jnp.concatenate(
        [head_w.T.astype(i32), rel_w.T.astype(i32), tail_w.T.astype(i32)],
        axis=0)
    idxs_e = jnp.stack([head, tail], axis=0)
    idxs_w = jnp.stack([head >> 4, tail >> 4, relation >> 4], axis=0)
    aux = jnp.stack([
        head & (_EV - 1), tail & (_EV - 1), relation & (_EV - 1),
        jnp.zeros_like(head), jnp.zeros_like(head), jnp.zeros_like(head),
        jnp.zeros_like(head), jnp.zeros_like(head),
    ], axis=1)

    word_s = _prescale(word_embedding, 1.0 / _W, 2000)
    rel_s = _prescale(relation_embedding, 1.0, 1000)
    proj_s = _prescale(relation_projection, 1.0, 1000)
    zeros_c = jnp.zeros((_C, _DP), jnp.float32)

    ebias16 = e_bias.reshape(-1, _EV)
    nr = r_bias.shape[0]
    rpad = (-nr) % _EV
    rbias16 = jnp.concatenate(
        [r_bias, jnp.zeros((rpad, 1), r_bias.dtype)], axis=0).reshape(-1, _EV)

    rl, pj, ah, ar, at_, hb, tb, rb = _sc_words(
        B, relation, idxs_w, words_t, zeros_c,
        rel_s, proj_s, word_s, ebias16, rbias16)
    eh, et = _sc_entity(B, idxs_e, entity_embedding)
    return _score(B, eh, et, rl, pj, ah, ar, at_, hb, tb, rb, aux)
